# baseline (speedup 1.0000x reference)
import jax
import jax.numpy as jnp
from jax.experimental import pallas as pl
from jax.experimental.pallas import tpu as pltpu
from jax.experimental.pallas import tpu_sc as plsc

N_FIELDS = 26
VOCAB = 100000
EMB_DIM = 18
PAD_DIM = 32
BATCH = 4096
CONCAT_DIM = N_FIELDS * EMB_DIM
WIDE_DIM = N_FIELDS * PAD_DIM
HIDDEN = 1024
OUT = 128
LEAKY_SLOPE = 0.01

NUM_INDICES = BATCH * N_FIELDS
GATHER_WINDOW = 128


def _sc_gather(table_pad, flat_idx):
    mesh = plsc.VectorSubcoreMesh(core_axis_name="core", subcore_axis_name="subcore")

    @pl.kernel(
        out_type=jax.ShapeDtypeStruct((NUM_INDICES, PAD_DIM), table_pad.dtype),
        mesh=mesh,
        compiler_params=pltpu.CompilerParams(use_tc_tiling_on_sc=False),
    )
    def gather_kernel(x_hbm, i_hbm, o_hbm):
        def body(i_vmem, o_vmem):
            pltpu.sync_copy(x_hbm.at[i_vmem.at[0]], o_vmem)

        pltpu.emit_pipeline(
            body,
            grid=(NUM_INDICES // GATHER_WINDOW,),
            in_specs=[pl.BlockSpec((1, GATHER_WINDOW), index_map=lambda i: (0, i))],
            out_specs=[pl.BlockSpec((GATHER_WINDOW, PAD_DIM), index_map=lambda i: (i, 0))],
            core_axis_name=("core", "subcore"),
            dimension_semantics=(pltpu.PARALLEL,),
        )(i_hbm, o_hbm)

    return gather_kernel(table_pad, flat_idx)


def _mlp_kernel(x_ref, w1_ref, b1_ref, w2_ref, b2_ref, o_ref):
    h = jnp.dot(x_ref[...], w1_ref[...], preferred_element_type=jnp.float32)
    h = h + b1_ref[...]
    h = jnp.where(h >= 0, h, h * LEAKY_SLOPE)
    o = jnp.dot(h.astype(jnp.bfloat16), w2_ref[...], preferred_element_type=jnp.float32)
    o_ref[...] = o + b2_ref[...]


def _mlp(embeds, W1e, b1, W2, b2):
    BB = 1024
    grid = (BATCH // BB,)
    return pl.pallas_call(
        _mlp_kernel,
        grid=grid,
        in_specs=[
            pl.BlockSpec((BB, WIDE_DIM), lambda i: (i, 0)),
            pl.BlockSpec((WIDE_DIM, HIDDEN), lambda i: (0, 0)),
            pl.BlockSpec((1, HIDDEN), lambda i: (0, 0)),
            pl.BlockSpec((HIDDEN, OUT), lambda i: (0, 0)),
            pl.BlockSpec((1, OUT), lambda i: (0, 0)),
        ],
        out_specs=pl.BlockSpec((BB, OUT), lambda i: (i, 0)),
        out_shape=jax.ShapeDtypeStruct((BATCH, OUT), jnp.float32),
    )(embeds, W1e, b1, W2, b2)


def kernel(categorical_data, tables, W1, b1, W2, b2):
    table_pad = jnp.pad(
        tables.astype(jnp.bfloat16).reshape(N_FIELDS * VOCAB, EMB_DIM),
        ((0, 0), (0, PAD_DIM - EMB_DIM)),
    )
    offsets = (jnp.arange(N_FIELDS, dtype=jnp.int32) * VOCAB)[None, :]
    flat_idx = (categorical_data + offsets).reshape(1, NUM_INDICES)
    wide = _sc_gather(table_pad, flat_idx)
    embeds = wide.reshape(BATCH, WIDE_DIM)
    W1e = jnp.pad(
        W1.astype(jnp.bfloat16).reshape(N_FIELDS, EMB_DIM, HIDDEN),
        ((0, 0), (0, PAD_DIM - EMB_DIM), (0, 0)),
    ).reshape(WIDE_DIM, HIDDEN)
    return _mlp(
        embeds,
        W1e,
        b1.reshape(1, HIDDEN),
        W2.astype(jnp.bfloat16),
        b2.reshape(1, OUT),
    )


# speedup vs baseline: 11.5510x; 11.5510x over previous
import jax
import jax.numpy as jnp
from jax.experimental import pallas as pl
from jax.experimental.pallas import tpu as pltpu
from jax.experimental.pallas import tpu_sc as plsc

N_FIELDS = 26
VOCAB = 100000
EMB_DIM = 18
PAD_DIM = 32
BATCH = 4096
WIDE_DIM = N_FIELDS * PAD_DIM
HIDDEN = 1024
OUT = 128
LEAKY_SLOPE = 0.01

NUM_INDICES = BATCH * N_FIELDS
GATHER_WINDOW = 128

BV = 1024
NBLK = -(-VOCAB // BV)
VSTORE = NBLK * BV
VS4 = VSTORE // 4


def _builder_kernel(t_ref, o_ref):
    x = t_ref[...]
    xt = jnp.transpose(x, (1, 2, 0))
    padded = jnp.concatenate(
        [xt, jnp.zeros((N_FIELDS, BV, PAD_DIM - EMB_DIM), jnp.float32)], axis=2
    )
    q = BV // 4
    o_ref[...] = jnp.concatenate(
        [padded[:, p * q:(p + 1) * q, :] for p in range(4)], axis=2
    )


def _build_table(t2):
    grid = (NBLK,)
    return pl.pallas_call(
        _builder_kernel,
        grid=grid,
        in_specs=[pl.BlockSpec((EMB_DIM, N_FIELDS, BV), lambda v: (0, 0, v))],
        out_specs=pl.BlockSpec((N_FIELDS, BV // 4, 128), lambda v: (0, v, 0)),
        out_shape=jax.ShapeDtypeStruct((N_FIELDS, VS4, 128), jnp.float32),
    )(t2)


def _sc_gather(table128, flat_idx):
    mesh = plsc.VectorSubcoreMesh(core_axis_name="core", subcore_axis_name="subcore")

    @pl.kernel(
        out_type=jax.ShapeDtypeStruct((NUM_INDICES, PAD_DIM), jnp.float32),
        mesh=mesh,
        compiler_params=pltpu.CompilerParams(use_tc_tiling_on_sc=False),
    )
    def gather_kernel(x_hbm, i_hbm, o_hbm):
        def body(i_vmem, o_vmem):
            pltpu.sync_copy(x_hbm.at[i_vmem.at[0]], o_vmem)

        pltpu.emit_pipeline(
            body,
            grid=(NUM_INDICES // GATHER_WINDOW,),
            in_specs=[pl.BlockSpec((1, GATHER_WINDOW), index_map=lambda i: (0, i))],
            out_specs=[pl.BlockSpec((GATHER_WINDOW, PAD_DIM), index_map=lambda i: (i, 0))],
            core_axis_name=("core", "subcore"),
            dimension_semantics=(pltpu.PARALLEL,),
        )(i_hbm, o_hbm)

    return gather_kernel(table128, flat_idx)


def _mlp_kernel(x_ref, w1_ref, b1_ref, w2_ref, b2_ref, o_ref):
    x = x_ref[...].astype(jnp.bfloat16)
    h = jnp.dot(x, w1_ref[...], preferred_element_type=jnp.float32)
    h = h + b1_ref[...]
    h = jnp.where(h >= 0, h, h * LEAKY_SLOPE)
    o = jnp.dot(h.astype(jnp.bfloat16), w2_ref[...], preferred_element_type=jnp.float32)
    o_ref[...] = o + b2_ref[...]


def _mlp(embeds, W1e, b1, W2, b2):
    BB = 1024
    grid = (BATCH // BB,)
    return pl.pallas_call(
        _mlp_kernel,
        grid=grid,
        in_specs=[
            pl.BlockSpec((BB, WIDE_DIM), lambda i: (i, 0)),
            pl.BlockSpec((WIDE_DIM, HIDDEN), lambda i: (0, 0)),
            pl.BlockSpec((1, HIDDEN), lambda i: (0, 0)),
            pl.BlockSpec((HIDDEN, OUT), lambda i: (0, 0)),
            pl.BlockSpec((1, OUT), lambda i: (0, 0)),
        ],
        out_specs=pl.BlockSpec((BB, OUT), lambda i: (i, 0)),
        out_shape=jax.ShapeDtypeStruct((BATCH, OUT), jnp.float32),
    )(embeds, W1e, b1, W2, b2)


def kernel(categorical_data, tables, W1, b1, W2, b2):
    t2 = jnp.transpose(tables, (2, 0, 1))
    table128 = _build_table(t2)
    table_rows = table128.reshape(N_FIELDS * VSTORE, PAD_DIM)
    q = BV // 4
    v = categorical_data
    loc = v % BV
    stored = (
        (jnp.arange(N_FIELDS, dtype=jnp.int32) * VS4)[None, :]
        + (v // BV) * (BV // 4)
        + (loc % q)
    ) * 4 + loc // q
    flat_idx = stored.reshape(1, NUM_INDICES)
    wide = _sc_gather(table_rows, flat_idx)
    embeds = wide.reshape(BATCH, WIDE_DIM)
    W1e = jnp.pad(
        W1.astype(jnp.bfloat16).reshape(N_FIELDS, EMB_DIM, HIDDEN),
        ((0, 0), (0, PAD_DIM - EMB_DIM), (0, 0)),
    ).reshape(WIDE_DIM, HIDDEN)
    return _mlp(
        embeds,
        W1e,
        b1.reshape(1, HIDDEN),
        W2.astype(jnp.bfloat16),
        b2.reshape(1, OUT),
    )


# speedup vs baseline: 28.7571x; 2.4896x over previous
import jax
import jax.numpy as jnp
from jax.experimental import pallas as pl
from jax.experimental.pallas import tpu as pltpu
from jax.experimental.pallas import tpu_sc as plsc

N_FIELDS = 26
VOCAB = 100000
EMB_DIM = 18
PAD_DIM = 32
BATCH = 4096
WIDE_DIM = N_FIELDS * PAD_DIM
HIDDEN = 1024
OUT = 128
LEAKY_SLOPE = 0.01

NUM_INDICES = BATCH * N_FIELDS
GATHER_WINDOW = 128

BV = 1024
NBLK = -(-VOCAB // BV)
VSTORE = NBLK * BV
VS4 = VSTORE // 4


def _builder_kernel(t_ref, o_ref):
    q = BV // 4
    jj = jax.lax.broadcasted_iota(jnp.int32, (EMB_DIM, 128), 0)
    cc = jax.lax.broadcasted_iota(jnp.int32, (EMB_DIM, 128), 1)
    sels = [
        (cc == 32 * p + jj).astype(jnp.float32) for p in range(4)
    ]
    dn = (((0,), (0,)), ((), ()))
    for f in range(N_FIELDS):
        acc = None
        for p in range(4):
            xs = t_ref[:, f, p * q:(p + 1) * q]
            d = jax.lax.dot_general(
                xs, sels[p], dn, preferred_element_type=jnp.float32
            )
            acc = d if acc is None else acc + d
        o_ref[f, :, :] = acc


def _build_table(t2):
    grid = (NBLK,)
    return pl.pallas_call(
        _builder_kernel,
        grid=grid,
        in_specs=[pl.BlockSpec((EMB_DIM, N_FIELDS, BV), lambda v: (0, 0, v))],
        out_specs=pl.BlockSpec((N_FIELDS, BV // 4, 128), lambda v: (0, v, 0)),
        out_shape=jax.ShapeDtypeStruct((N_FIELDS, VS4, 128), jnp.float32),
    )(t2)


def _sc_gather(table128, flat_idx):
    mesh = plsc.VectorSubcoreMesh(core_axis_name="core", subcore_axis_name="subcore")

    @pl.kernel(
        out_type=jax.ShapeDtypeStruct((NUM_INDICES, PAD_DIM), jnp.float32),
        mesh=mesh,
        compiler_params=pltpu.CompilerParams(use_tc_tiling_on_sc=False),
    )
    def gather_kernel(x_hbm, i_hbm, o_hbm):
        def body(i_vmem, o_vmem):
            pltpu.sync_copy(x_hbm.at[i_vmem.at[0]], o_vmem)

        pltpu.emit_pipeline(
            body,
            grid=(NUM_INDICES // GATHER_WINDOW,),
            in_specs=[pl.BlockSpec((1, GATHER_WINDOW), index_map=lambda i: (0, i))],
            out_specs=[pl.BlockSpec((GATHER_WINDOW, PAD_DIM), index_map=lambda i: (i, 0))],
            core_axis_name=("core", "subcore"),
            dimension_semantics=(pltpu.PARALLEL,),
        )(i_hbm, o_hbm)

    return gather_kernel(table128, flat_idx)


def _mlp_kernel(x_ref, w1_ref, b1_ref, w2_ref, b2_ref, o_ref):
    x = x_ref[...].astype(jnp.bfloat16)
    h = jnp.dot(x, w1_ref[...], preferred_element_type=jnp.float32)
    h = h + b1_ref[...]
    h = jnp.where(h >= 0, h, h * LEAKY_SLOPE)
    o = jnp.dot(h.astype(jnp.bfloat16), w2_ref[...], preferred_element_type=jnp.float32)
    o_ref[...] = o + b2_ref[...]


def _mlp(embeds, W1e, b1, W2, b2):
    BB = 1024
    grid = (BATCH // BB,)
    return pl.pallas_call(
        _mlp_kernel,
        grid=grid,
        in_specs=[
            pl.BlockSpec((BB, WIDE_DIM), lambda i: (i, 0)),
            pl.BlockSpec((WIDE_DIM, HIDDEN), lambda i: (0, 0)),
            pl.BlockSpec((1, HIDDEN), lambda i: (0, 0)),
            pl.BlockSpec((HIDDEN, OUT), lambda i: (0, 0)),
            pl.BlockSpec((1, OUT), lambda i: (0, 0)),
        ],
        out_specs=pl.BlockSpec((BB, OUT), lambda i: (i, 0)),
        out_shape=jax.ShapeDtypeStruct((BATCH, OUT), jnp.float32),
    )(embeds, W1e, b1, W2, b2)


def kernel(categorical_data, tables, W1, b1, W2, b2):
    t2 = jnp.transpose(tables, (2, 0, 1))
    table128 = _build_table(t2)
    table_rows = table128.reshape(N_FIELDS * VSTORE, PAD_DIM)
    q = BV // 4
    v = categorical_data
    loc = v % BV
    stored = (
        (jnp.arange(N_FIELDS, dtype=jnp.int32) * VS4)[None, :]
        + (v // BV) * (BV // 4)
        + (loc % q)
    ) * 4 + loc // q
    flat_idx = stored.reshape(1, NUM_INDICES)
    wide = _sc_gather(table_rows, flat_idx)
    embeds = wide.reshape(BATCH, WIDE_DIM)
    W1e = jnp.pad(
        W1.astype(jnp.bfloat16).reshape(N_FIELDS, EMB_DIM, HIDDEN),
        ((0, 0), (0, PAD_DIM - EMB_DIM), (0, 0)),
    ).reshape(WIDE_DIM, HIDDEN)
    return _mlp(
        embeds,
        W1e,
        b1.reshape(1, HIDDEN),
        W2.astype(jnp.bfloat16),
        b2.reshape(1, OUT),
    )


# speedup vs baseline: 44.9660x; 1.5636x over previous
import jax
import jax.numpy as jnp
from jax.experimental import pallas as pl
from jax.experimental.pallas import tpu as pltpu
from jax.experimental.pallas import tpu_sc as plsc

N_FIELDS = 26
VOCAB = 100000
EMB_DIM = 18
PAD_DIM = 32
BATCH = 4096
WIDE_DIM = N_FIELDS * PAD_DIM
HIDDEN = 1024
OUT = 128
LEAKY_SLOPE = 0.01

NUM_INDICES = BATCH * N_FIELDS
GATHER_WINDOW = 128

BV = 1024
NBLK = -(-VOCAB // BV)
VSTORE = NBLK * BV
VS4 = VSTORE // 4


def _builder_kernel(t_ref, o_ref):
    q = BV // 4
    kk = jax.lax.broadcasted_iota(jnp.int32, (4 * EMB_DIM, 128), 0)
    cc = jax.lax.broadcasted_iota(jnp.int32, (4 * EMB_DIM, 128), 1)
    sel = (cc == 32 * (kk // EMB_DIM) + kk % EMB_DIM).astype(jnp.bfloat16)
    dn = (((0,), (0,)), ((), ()))
    for f in range(N_FIELDS):
        xs = jnp.concatenate(
            [t_ref[:, f, p * q:(p + 1) * q] for p in range(4)], axis=0
        ).astype(jnp.bfloat16)
        o_ref[f, :, :] = jax.lax.dot_general(
            xs, sel, dn, preferred_element_type=jnp.float32
        )


def _build_table(t2):
    grid = (NBLK,)
    return pl.pallas_call(
        _builder_kernel,
        grid=grid,
        in_specs=[pl.BlockSpec((EMB_DIM, N_FIELDS, BV), lambda v: (0, 0, v))],
        out_specs=pl.BlockSpec((N_FIELDS, BV // 4, 128), lambda v: (0, v, 0)),
        out_shape=jax.ShapeDtypeStruct((N_FIELDS, VS4, 128), jnp.float32),
    )(t2)


def _sc_gather(table128, flat_idx):
    mesh = plsc.VectorSubcoreMesh(core_axis_name="core", subcore_axis_name="subcore")

    @pl.kernel(
        out_type=jax.ShapeDtypeStruct((NUM_INDICES, PAD_DIM), jnp.float32),
        mesh=mesh,
        compiler_params=pltpu.CompilerParams(use_tc_tiling_on_sc=False),
    )
    def gather_kernel(x_hbm, i_hbm, o_hbm):
        def body(i_vmem, o_vmem):
            pltpu.sync_copy(x_hbm.at[i_vmem.at[0]], o_vmem)

        pltpu.emit_pipeline(
            body,
            grid=(NUM_INDICES // GATHER_WINDOW,),
            in_specs=[pl.BlockSpec((1, GATHER_WINDOW), index_map=lambda i: (0, i))],
            out_specs=[pl.BlockSpec((GATHER_WINDOW, PAD_DIM), index_map=lambda i: (i, 0))],
            core_axis_name=("core", "subcore"),
            dimension_semantics=(pltpu.PARALLEL,),
        )(i_hbm, o_hbm)

    return gather_kernel(table128, flat_idx)


def _mlp_kernel(x_ref, w1_ref, b1_ref, w2_ref, b2_ref, o_ref):
    x = x_ref[...].astype(jnp.bfloat16)
    h = jnp.dot(x, w1_ref[...], preferred_element_type=jnp.float32)
    h = h + b1_ref[...]
    h = jnp.where(h >= 0, h, h * LEAKY_SLOPE)
    o = jnp.dot(h.astype(jnp.bfloat16), w2_ref[...], preferred_element_type=jnp.float32)
    o_ref[...] = o + b2_ref[...]


def _mlp(embeds, W1e, b1, W2, b2):
    BB = 1024
    grid = (BATCH // BB,)
    return pl.pallas_call(
        _mlp_kernel,
        grid=grid,
        in_specs=[
            pl.BlockSpec((BB, WIDE_DIM), lambda i: (i, 0)),
            pl.BlockSpec((WIDE_DIM, HIDDEN), lambda i: (0, 0)),
            pl.BlockSpec((1, HIDDEN), lambda i: (0, 0)),
            pl.BlockSpec((HIDDEN, OUT), lambda i: (0, 0)),
            pl.BlockSpec((1, OUT), lambda i: (0, 0)),
        ],
        out_specs=pl.BlockSpec((BB, OUT), lambda i: (i, 0)),
        out_shape=jax.ShapeDtypeStruct((BATCH, OUT), jnp.float32),
    )(embeds, W1e, b1, W2, b2)


def kernel(categorical_data, tables, W1, b1, W2, b2):
    t2 = jnp.transpose(tables, (2, 0, 1))
    table128 = _build_table(t2)
    table_rows = table128.reshape(N_FIELDS * VSTORE, PAD_DIM)
    q = BV // 4
    v = categorical_data
    loc = v % BV
    stored = (
        (jnp.arange(N_FIELDS, dtype=jnp.int32) * VS4)[None, :]
        + (v // BV) * (BV // 4)
        + (loc % q)
    ) * 4 + loc // q
    flat_idx = stored.reshape(1, NUM_INDICES)
    wide = _sc_gather(table_rows, flat_idx)
    embeds = wide.reshape(BATCH, WIDE_DIM)
    W1e = jnp.pad(
        W1.astype(jnp.bfloat16).reshape(N_FIELDS, EMB_DIM, HIDDEN),
        ((0, 0), (0, PAD_DIM - EMB_DIM), (0, 0)),
    ).reshape(WIDE_DIM, HIDDEN)
    return _mlp(
        embeds,
        W1e,
        b1.reshape(1, HIDDEN),
        W2.astype(jnp.bfloat16),
        b2.reshape(1, OUT),
    )


# speedup vs baseline: 49.7682x; 1.1068x over previous
import jax
import jax.numpy as jnp
from jax.experimental import pallas as pl
from jax.experimental.pallas import tpu as pltpu
from jax.experimental.pallas import tpu_sc as plsc

N_FIELDS = 26
VOCAB = 100000
EMB_DIM = 18
PAD_DIM = 32
BATCH = 4096
WIDE_DIM = N_FIELDS * PAD_DIM
HIDDEN = 1024
OUT = 128
LEAKY_SLOPE = 0.01

NUM_INDICES = BATCH * N_FIELDS
GATHER_WINDOW = 128

BV = 4096
NBLK = -(-VOCAB // BV)
VSTORE = NBLK * BV
VS4 = VSTORE // 4


def _builder_kernel(t_ref, o_ref):
    q = BV // 4
    kk = jax.lax.broadcasted_iota(jnp.int32, (4 * EMB_DIM, 128), 0)
    cc = jax.lax.broadcasted_iota(jnp.int32, (4 * EMB_DIM, 128), 1)
    sel = (cc == 32 * (kk // EMB_DIM) + kk % EMB_DIM).astype(jnp.bfloat16)
    dn = (((0,), (0,)), ((), ()))
    for f in range(N_FIELDS):
        xs = jnp.concatenate(
            [t_ref[:, f, p * q:(p + 1) * q] for p in range(4)], axis=0
        ).astype(jnp.bfloat16)
        o_ref[f, :, :] = jax.lax.dot_general(
            xs, sel, dn, preferred_element_type=jnp.float32
        )


def _build_table(t2):
    grid = (NBLK,)
    return pl.pallas_call(
        _builder_kernel,
        grid=grid,
        in_specs=[pl.BlockSpec((EMB_DIM, N_FIELDS, BV), lambda v: (0, 0, v))],
        out_specs=pl.BlockSpec((N_FIELDS, BV // 4, 128), lambda v: (0, v, 0)),
        out_shape=jax.ShapeDtypeStruct((N_FIELDS, VS4, 128), jnp.float32),
    )(t2)


def _sc_gather(table128, flat_idx):
    mesh = plsc.VectorSubcoreMesh(core_axis_name="core", subcore_axis_name="subcore")

    @pl.kernel(
        out_type=jax.ShapeDtypeStruct((NUM_INDICES, PAD_DIM), jnp.float32),
        mesh=mesh,
        compiler_params=pltpu.CompilerParams(use_tc_tiling_on_sc=False),
    )
    def gather_kernel(x_hbm, i_hbm, o_hbm):
        def body(i_vmem, o_vmem):
            pltpu.sync_copy(x_hbm.at[i_vmem.at[0]], o_vmem)

        pltpu.emit_pipeline(
            body,
            grid=(NUM_INDICES // GATHER_WINDOW,),
            in_specs=[pl.BlockSpec((1, GATHER_WINDOW), index_map=lambda i: (0, i))],
            out_specs=[pl.BlockSpec((GATHER_WINDOW, PAD_DIM), index_map=lambda i: (i, 0))],
            core_axis_name=("core", "subcore"),
            dimension_semantics=(pltpu.PARALLEL,),
        )(i_hbm, o_hbm)

    return gather_kernel(table128, flat_idx)


def _mlp_kernel(x_ref, w1_ref, b1_ref, w2_ref, b2_ref, o_ref):
    x = x_ref[...].astype(jnp.bfloat16)
    h = jnp.dot(x, w1_ref[...], preferred_element_type=jnp.float32)
    h = h + b1_ref[...]
    h = jnp.where(h >= 0, h, h * LEAKY_SLOPE)
    o = jnp.dot(h.astype(jnp.bfloat16), w2_ref[...], preferred_element_type=jnp.float32)
    o_ref[...] = o + b2_ref[...]


def _mlp(embeds, W1e, b1, W2, b2):
    BB = 1024
    grid = (BATCH // BB,)
    return pl.pallas_call(
        _mlp_kernel,
        grid=grid,
        in_specs=[
            pl.BlockSpec((BB, WIDE_DIM), lambda i: (i, 0)),
            pl.BlockSpec((WIDE_DIM, HIDDEN), lambda i: (0, 0)),
            pl.BlockSpec((1, HIDDEN), lambda i: (0, 0)),
            pl.BlockSpec((HIDDEN, OUT), lambda i: (0, 0)),
            pl.BlockSpec((1, OUT), lambda i: (0, 0)),
        ],
        out_specs=pl.BlockSpec((BB, OUT), lambda i: (i, 0)),
        out_shape=jax.ShapeDtypeStruct((BATCH, OUT), jnp.float32),
    )(embeds, W1e, b1, W2, b2)


def kernel(categorical_data, tables, W1, b1, W2, b2):
    t2 = jnp.transpose(tables, (2, 0, 1))
    table128 = _build_table(t2)
    table_rows = table128.reshape(N_FIELDS * VSTORE, PAD_DIM)
    q = BV // 4
    v = categorical_data
    loc = v % BV
    stored = (
        (jnp.arange(N_FIELDS, dtype=jnp.int32) * VS4)[None, :]
        + (v // BV) * (BV // 4)
        + (loc % q)
    ) * 4 + loc // q
    flat_idx = stored.reshape(1, NUM_INDICES)
    wide = _sc_gather(table_rows, flat_idx)
    embeds = wide.reshape(BATCH, WIDE_DIM)
    W1e = jnp.pad(
        W1.astype(jnp.bfloat16).reshape(N_FIELDS, EMB_DIM, HIDDEN),
        ((0, 0), (0, PAD_DIM - EMB_DIM), (0, 0)),
    ).reshape(WIDE_DIM, HIDDEN)
    return _mlp(
        embeds,
        W1e,
        b1.reshape(1, HIDDEN),
        W2.astype(jnp.bfloat16),
        b2.reshape(1, OUT),
    )


# speedup vs baseline: 62.0901x; 1.2476x over previous
import jax
import jax.numpy as jnp
from jax.experimental import pallas as pl
from jax.experimental.pallas import tpu as pltpu
from jax.experimental.pallas import tpu_sc as plsc

N_FIELDS = 26
VOCAB = 100000
EMB_DIM = 18
PACK_DIM = 16
BATCH = 4096
WIDE_DIM = N_FIELDS * PACK_DIM
HIDDEN = 1024
OUT = 128
LEAKY_SLOPE = 0.01

NUM_INDICES = BATCH * N_FIELDS
GATHER_WINDOW = 128

BV = 4096
NBLK = -(-VOCAB // BV)
VSTORE = NBLK * BV
VS8 = VSTORE // 8
FOLD = 8


def _builder_kernel(t_ref, o_ref):
    q8 = BV // FOLD
    kk = jax.lax.broadcasted_iota(jnp.int32, (FOLD * EMB_DIM, 128), 0)
    cc = jax.lax.broadcasted_iota(jnp.int32, (FOLD * EMB_DIM, 128), 1)
    same_p = (cc // PACK_DIM) == (kk // EMB_DIM)
    sel_lo = (same_p & ((kk % EMB_DIM) == 2 * (cc % PACK_DIM))).astype(jnp.bfloat16)
    sel_hi = (same_p & ((kk % EMB_DIM) == 2 * (cc % PACK_DIM) + 1)).astype(jnp.bfloat16)
    dn = (((0,), (0,)), ((), ()))
    for f in range(N_FIELDS):
        xs = jnp.concatenate(
            [t_ref[:, f, p * q8:(p + 1) * q8] for p in range(FOLD)], axis=0
        ).astype(jnp.bfloat16)
        d_lo = jax.lax.dot_general(
            xs, sel_lo, dn, preferred_element_type=jnp.float32
        )
        d_hi = jax.lax.dot_general(
            xs, sel_hi, dn, preferred_element_type=jnp.float32
        )
        bits_lo = jax.lax.shift_right_logical(
            pltpu.bitcast(d_lo, jnp.int32), 16
        )
        bits_hi = pltpu.bitcast(d_hi, jnp.int32) & jnp.int32(-65536)
        o_ref[f, :, :] = bits_hi | bits_lo


def _build_table(t2):
    grid = (NBLK,)
    return pl.pallas_call(
        _builder_kernel,
        grid=grid,
        in_specs=[pl.BlockSpec((EMB_DIM, N_FIELDS, BV), lambda v: (0, 0, v))],
        out_specs=pl.BlockSpec((N_FIELDS, BV // FOLD, 128), lambda v: (0, v, 0)),
        out_shape=jax.ShapeDtypeStruct((N_FIELDS, VS8, 128), jnp.int32),
    )(t2)


def _sc_gather(table_rows, flat_idx):
    mesh = plsc.VectorSubcoreMesh(core_axis_name="core", subcore_axis_name="subcore")

    @pl.kernel(
        out_type=jax.ShapeDtypeStruct((NUM_INDICES, PACK_DIM), jnp.int32),
        mesh=mesh,
        compiler_params=pltpu.CompilerParams(use_tc_tiling_on_sc=False),
    )
    def gather_kernel(x_hbm, i_hbm, o_hbm):
        def body(i_vmem, o_vmem):
            pltpu.sync_copy(x_hbm.at[i_vmem.at[0]], o_vmem)

        pltpu.emit_pipeline(
            body,
            grid=(NUM_INDICES // GATHER_WINDOW,),
            in_specs=[pl.BlockSpec((1, GATHER_WINDOW), index_map=lambda i: (0, i))],
            out_specs=[pl.BlockSpec((GATHER_WINDOW, PACK_DIM), index_map=lambda i: (i, 0))],
            core_axis_name=("core", "subcore"),
            dimension_semantics=(pltpu.PARALLEL,),
        )(i_hbm, o_hbm)

    return gather_kernel(table_rows, flat_idx)


def _mlp_kernel(x_ref, w1e_ref, w1o_ref, b1_ref, w2_ref, b2_ref, o_ref):
    xi = x_ref[...]
    lo = pltpu.bitcast(jax.lax.shift_left(xi, 16), jnp.float32).astype(jnp.bfloat16)
    hi = pltpu.bitcast(xi & jnp.int32(-65536), jnp.float32).astype(jnp.bfloat16)
    h = jax.lax.dot_general(
        lo, w1e_ref[...], (((1,), (0,)), ((), ())),
        preferred_element_type=jnp.float32,
    )
    h = h + jax.lax.dot_general(
        hi, w1o_ref[...], (((1,), (0,)), ((), ())),
        preferred_element_type=jnp.float32,
    )
    h = h + b1_ref[...]
    h = jnp.where(h >= 0, h, h * LEAKY_SLOPE)
    o = jnp.dot(h.astype(jnp.bfloat16), w2_ref[...], preferred_element_type=jnp.float32)
    o_ref[...] = o + b2_ref[...]


def _mlp(embeds, W1e, W1o, b1, W2, b2):
    BB = 1024
    grid = (BATCH // BB,)
    return pl.pallas_call(
        _mlp_kernel,
        grid=grid,
        in_specs=[
            pl.BlockSpec((BB, WIDE_DIM), lambda i: (i, 0)),
            pl.BlockSpec((WIDE_DIM, HIDDEN), lambda i: (0, 0)),
            pl.BlockSpec((WIDE_DIM, HIDDEN), lambda i: (0, 0)),
            pl.BlockSpec((1, HIDDEN), lambda i: (0, 0)),
            pl.BlockSpec((HIDDEN, OUT), lambda i: (0, 0)),
            pl.BlockSpec((1, OUT), lambda i: (0, 0)),
        ],
        out_specs=pl.BlockSpec((BB, OUT), lambda i: (i, 0)),
        out_shape=jax.ShapeDtypeStruct((BATCH, OUT), jnp.float32),
    )(embeds, W1e, W1o, b1, W2, b2)


def kernel(categorical_data, tables, W1, b1, W2, b2):
    t2 = jnp.transpose(tables, (2, 0, 1))
    table128 = _build_table(t2)
    table_rows = table128.reshape(N_FIELDS * VSTORE, PACK_DIM)
    q8 = BV // FOLD
    v = categorical_data
    loc = v % BV
    stored = 8 * (
        (jnp.arange(N_FIELDS, dtype=jnp.int32) * VS8)[None, :]
        + (v // BV) * (BV // FOLD)
        + (loc % q8)
    ) + loc // q8
    flat_idx = stored.reshape(1, NUM_INDICES)
    wide = _sc_gather(table_rows, flat_idx)
    embeds = wide.reshape(BATCH, WIDE_DIM)
    W1r = W1.astype(jnp.bfloat16).reshape(N_FIELDS, EMB_DIM, HIDDEN)
    W1e = jnp.pad(W1r[:, 0::2, :], ((0, 0), (0, PACK_DIM - 9), (0, 0))).reshape(
        WIDE_DIM, HIDDEN
    )
    W1o = jnp.pad(W1r[:, 1::2, :], ((0, 0), (0, PACK_DIM - 9), (0, 0))).reshape(
        WIDE_DIM, HIDDEN
    )
    return _mlp(
        embeds,
        W1e,
        W1o,
        b1.reshape(1, HIDDEN),
        W2.astype(jnp.bfloat16),
        b2.reshape(1, OUT),
    )


# speedup vs baseline: 62.8231x; 1.0118x over previous
import jax
import jax.numpy as jnp
from jax.experimental import pallas as pl
from jax.experimental.pallas import tpu as pltpu
from jax.experimental.pallas import tpu_sc as plsc

N_FIELDS = 26
VOCAB = 100000
EMB_DIM = 18
PACK_DIM = 16
BATCH = 4096
WIDE_DIM = N_FIELDS * PACK_DIM
HIDDEN = 1024
OUT = 128
LEAKY_SLOPE = 0.01

NUM_INDICES = BATCH * N_FIELDS
GATHER_WINDOW = 128

BV = 4096
NBLK = -(-VOCAB // BV)
VSTORE = NBLK * BV
VS8 = VSTORE // 8
FOLD = 8


def _builder_kernel(t_ref, o_ref):
    q8 = BV // FOLD
    kk = jax.lax.broadcasted_iota(jnp.int32, (FOLD * EMB_DIM, 256), 0)
    cc = jax.lax.broadcasted_iota(jnp.int32, (FOLD * EMB_DIM, 256), 1)
    c128 = cc % 128
    same_p = (c128 // PACK_DIM) == (kk // EMB_DIM)
    sel = (
        same_p & ((kk % EMB_DIM) == 2 * (c128 % PACK_DIM) + cc // 128)
    ).astype(jnp.bfloat16)
    dn = (((0,), (0,)), ((), ()))
    for f in range(N_FIELDS):
        xs = jnp.concatenate(
            [t_ref[:, f, p * q8:(p + 1) * q8] for p in range(FOLD)], axis=0
        ).astype(jnp.bfloat16)
        d = jax.lax.dot_general(
            xs, sel, dn, preferred_element_type=jnp.float32
        )
        d_lo = d[:, :128]
        d_hi = d[:, 128:]
        bits_lo = jax.lax.shift_right_logical(
            pltpu.bitcast(d_lo, jnp.int32), 16
        )
        bits_hi = pltpu.bitcast(d_hi, jnp.int32) & jnp.int32(-65536)
        o_ref[f, :, :] = bits_hi | bits_lo


def _build_table(t2):
    grid = (NBLK,)
    return pl.pallas_call(
        _builder_kernel,
        grid=grid,
        in_specs=[pl.BlockSpec((EMB_DIM, N_FIELDS, BV), lambda v: (0, 0, v))],
        out_specs=pl.BlockSpec((N_FIELDS, BV // FOLD, 128), lambda v: (0, v, 0)),
        out_shape=jax.ShapeDtypeStruct((N_FIELDS, VS8, 128), jnp.int32),
    )(t2)


def _sc_gather(table_rows, flat_idx):
    mesh = plsc.VectorSubcoreMesh(core_axis_name="core", subcore_axis_name="subcore")

    @pl.kernel(
        out_type=jax.ShapeDtypeStruct((NUM_INDICES, PACK_DIM), jnp.int32),
        mesh=mesh,
        compiler_params=pltpu.CompilerParams(use_tc_tiling_on_sc=False),
    )
    def gather_kernel(x_hbm, i_hbm, o_hbm):
        def body(i_vmem, o_vmem):
            pltpu.sync_copy(x_hbm.at[i_vmem.at[0]], o_vmem)

        pltpu.emit_pipeline(
            body,
            grid=(NUM_INDICES // GATHER_WINDOW,),
            in_specs=[pl.BlockSpec((1, GATHER_WINDOW), index_map=lambda i: (0, i))],
            out_specs=[pl.BlockSpec((GATHER_WINDOW, PACK_DIM), index_map=lambda i: (i, 0))],
            core_axis_name=("core", "subcore"),
            dimension_semantics=(pltpu.PARALLEL,),
        )(i_hbm, o_hbm)

    return gather_kernel(table_rows, flat_idx)


def _mlp_kernel(x_ref, w1e_ref, w1o_ref, b1_ref, w2_ref, b2_ref, o_ref):
    xi = x_ref[...]
    lo = pltpu.bitcast(jax.lax.shift_left(xi, 16), jnp.float32).astype(jnp.bfloat16)
    hi = pltpu.bitcast(xi & jnp.int32(-65536), jnp.float32).astype(jnp.bfloat16)
    h = jax.lax.dot_general(
        lo, w1e_ref[...], (((1,), (0,)), ((), ())),
        preferred_element_type=jnp.float32,
    )
    h = h + jax.lax.dot_general(
        hi, w1o_ref[...], (((1,), (0,)), ((), ())),
        preferred_element_type=jnp.float32,
    )
    h = h + b1_ref[...]
    h = jnp.where(h >= 0, h, h * LEAKY_SLOPE)
    o = jnp.dot(h.astype(jnp.bfloat16), w2_ref[...], preferred_element_type=jnp.float32)
    o_ref[...] = o + b2_ref[...]


def _mlp(embeds, W1e, W1o, b1, W2, b2):
    BB = 1024
    grid = (BATCH // BB,)
    return pl.pallas_call(
        _mlp_kernel,
        grid=grid,
        in_specs=[
            pl.BlockSpec((BB, WIDE_DIM), lambda i: (i, 0)),
            pl.BlockSpec((WIDE_DIM, HIDDEN), lambda i: (0, 0)),
            pl.BlockSpec((WIDE_DIM, HIDDEN), lambda i: (0, 0)),
            pl.BlockSpec((1, HIDDEN), lambda i: (0, 0)),
            pl.BlockSpec((HIDDEN, OUT), lambda i: (0, 0)),
            pl.BlockSpec((1, OUT), lambda i: (0, 0)),
        ],
        out_specs=pl.BlockSpec((BB, OUT), lambda i: (i, 0)),
        out_shape=jax.ShapeDtypeStruct((BATCH, OUT), jnp.float32),
    )(embeds, W1e, W1o, b1, W2, b2)


def kernel(categorical_data, tables, W1, b1, W2, b2):
    t2 = jnp.transpose(tables, (2, 0, 1))
    table128 = _build_table(t2)
    table_rows = table128.reshape(N_FIELDS * VSTORE, PACK_DIM)
    q8 = BV // FOLD
    v = categorical_data
    loc = v % BV
    stored = 8 * (
        (jnp.arange(N_FIELDS, dtype=jnp.int32) * VS8)[None, :]
        + (v // BV) * (BV // FOLD)
        + (loc % q8)
    ) + loc // q8
    flat_idx = stored.reshape(1, NUM_INDICES)
    wide = _sc_gather(table_rows, flat_idx)
    embeds = wide.reshape(BATCH, WIDE_DIM)
    W1r = W1.astype(jnp.bfloat16).reshape(N_FIELDS, EMB_DIM, HIDDEN)
    W1e = jnp.pad(W1r[:, 0::2, :], ((0, 0), (0, PACK_DIM - 9), (0, 0))).reshape(
        WIDE_DIM, HIDDEN
    )
    W1o = jnp.pad(W1r[:, 1::2, :], ((0, 0), (0, PACK_DIM - 9), (0, 0))).reshape(
        WIDE_DIM, HIDDEN
    )
    return _mlp(
        embeds,
        W1e,
        W1o,
        b1.reshape(1, HIDDEN),
        W2.astype(jnp.bfloat16),
        b2.reshape(1, OUT),
    )
